# Initial kernel scaffold; baseline (speedup 1.0000x reference)
import jax
import jax.numpy as jnp
from jax.experimental import pallas as pl


def kernel(attributes, text_feats, Vgs):
    raise NotImplementedError("write your pallas kernel here")


# speedup vs baseline: 9.2627x; 9.2627x over previous
import jax
import jax.numpy as jnp
from jax.experimental import pallas as pl
from jax.experimental.pallas import tpu as pltpu

_EPS = 1e-8
_NSEG = 8


def _body(attr_ref, x_ref, vg_ref, out_ref, acc_ref, cnt_ref):
    b = pl.program_id(0)

    @pl.when(b == 0)
    def _():
        acc_ref[0] = 0.0
        cnt_ref[0] = 0

    attr = attr_ref[0, 0, :]
    x = x_ref[0]
    vg = vg_ref[0, 0]

    seg_ids = jax.lax.broadcasted_iota(jnp.int32, (_NSEG, attr.shape[0]), 0)
    mask = (seg_ids == attr[None, :]).astype(jnp.float32)
    seg_sums = jnp.dot(mask, x, preferred_element_type=jnp.float32)
    counts = jnp.sum(mask, axis=1, keepdims=True)
    mean = seg_sums / counts

    num = jnp.sum(mean * vg[None, :], axis=1, keepdims=True)
    norm_m = jnp.sqrt(jnp.sum(mean * mean, axis=1, keepdims=True))
    norm_vg = jnp.sqrt(jnp.sum(vg * vg))
    denom = jnp.maximum(norm_vg, _EPS) * jnp.maximum(norm_m, _EPS)
    cos = num / denom

    max_attr = jnp.max(attr)
    idx = jax.lax.broadcasted_iota(jnp.int32, (_NSEG, 1), 0)
    valid = (idx >= 1) & (idx <= max_attr)
    cs = jnp.sum(jnp.where(valid, cos, 0.0)) / max_attr.astype(jnp.float32)
    has_any = max_attr > 0

    acc_ref[0] += jnp.where(has_any, 1.0 - cs, 0.0)
    cnt_ref[0] += has_any.astype(jnp.int32)

    @pl.when(b == pl.num_programs(0) - 1)
    def _():
        out_ref[0, 0] = acc_ref[0] / cnt_ref[0].astype(jnp.float32)


def kernel(attributes, text_feats, Vgs):
    B, T = attributes.shape
    D = text_feats.shape[-1]
    attr3 = attributes.astype(jnp.int32).reshape(B, 1, T)
    vgs3 = Vgs.reshape(B, 1, D)
    out = pl.pallas_call(
        _body,
        grid=(B,),
        in_specs=[
            pl.BlockSpec((1, 1, T), lambda b: (b, 0, 0)),
            pl.BlockSpec((1, T, D), lambda b: (b, 0, 0)),
            pl.BlockSpec((1, 1, D), lambda b: (b, 0, 0)),
        ],
        out_specs=pl.BlockSpec(memory_space=pltpu.SMEM),
        out_shape=jax.ShapeDtypeStruct((1, 1), jnp.float32),
        scratch_shapes=[
            pltpu.SMEM((1,), jnp.float32),
            pltpu.SMEM((1,), jnp.int32),
        ],
    )(attr3, text_feats, vgs3)
    return out[0, 0]
